# baseline (speedup 1.0000x reference)
import functools

import jax
import jax.numpy as jnp
from jax import lax
from jax.experimental import pallas as pl
from jax.experimental.pallas import tpu as pltpu
from jax.experimental.pallas import tpu_sc as plsc


def _sc_embed_pool(idx_all, table):
    NI, L = idx_all.shape
    _, E = table.shape
    mesh = plsc.VectorSubcoreMesh(core_axis_name="c", subcore_axis_name="s")
    NW = mesh.num_cores * mesh.num_subcores
    IPW = NI // NW
    EC = E // 16
    PI = 2
    NP = IPW // PI
    NBUF = 4
    GPAD = 4
    GL = PI * L
    idx_grp = idx_all.reshape(NI // PI, GL)

    offs = list(range(0, GL - 15, 16))
    if GL % 16:
        offs.append(GL - 16)

    @functools.partial(
        pl.kernel,
        out_type=jax.ShapeDtypeStruct((NI, E), jnp.float32),
        mesh=mesh,
        scratch_types=[
            pltpu.VMEM((NP + GPAD, GL), jnp.int32),
            pltpu.VMEM((NBUF, GL, E), jnp.float32),
            pltpu.VMEM((IPW, E), jnp.float32),
            pltpu.SemaphoreType.DMA((NBUF,)),
        ],
        compiler_params=pltpu.CompilerParams(use_tc_tiling_on_sc=False),
    )
    def k(idx_hbm, tab_hbm, out_hbm, idx_v, rows_v, out_v, sems):
        wid = lax.axis_index("s") * mesh.num_cores + lax.axis_index("c")
        base = wid * IPW
        zero16 = jnp.zeros((16,), jnp.int32)
        for r in range(NP, NP + GPAD):
            for off in offs:
                idx_v[r, pl.ds(off, 16)] = zero16
        pltpu.sync_copy(idx_hbm.at[pl.ds(wid * NP, NP)],
                        idx_v.at[pl.ds(0, NP)])

        def start(p, buf):
            pltpu.async_copy(tab_hbm.at[idx_v.at[p]],
                             rows_v.at[buf], sems.at[buf])

        def wait(buf):
            pltpu.make_async_copy(tab_hbm.at[idx_v.at[0]],
                                  rows_v.at[buf], sems.at[buf]).wait()

        def accum(p, buf):
            for j in range(PI):
                r0 = j * L
                for c in range(EC):
                    sl = pl.ds(c * 16, 16)
                    a = rows_v[buf, r0, sl] + rows_v[buf, r0 + 1, sl]
                    b = rows_v[buf, r0 + 2, sl] + rows_v[buf, r0 + 3, sl]
                    for l in range(4, L - 1, 2):
                        a = a + rows_v[buf, r0 + l, sl]
                        b = b + rows_v[buf, r0 + l + 1, sl]
                    if L % 2:
                        a = a + rows_v[buf, r0 + L - 1, sl]
                    out_v[p * PI + j, sl] = a + b

        for p in range(NBUF - 1):
            start(p, p)

        @pl.loop(0, NP, step=NBUF)
        def _(p):
            for k in range(NBUF):
                start(p + k + NBUF - 1, (k + NBUF - 1) % NBUF)
                wait(k)
                accum(p + k, k)

        for k in range(NBUF - 1):
            wait(k)
        pltpu.sync_copy(out_v, out_hbm.at[pl.ds(base, IPW)])

    return k(idx_grp, table)


def _logsig(x):
    return jnp.minimum(x, 0.0) - jnp.log(1.0 + jnp.exp(-jnp.abs(x)))


def _tc_score(rel, v1, v2, wt, b2, relt, hid):
    B, E = v1.shape
    H = hid
    Bb = 256
    NB = B // Bb
    CH = 32
    NCH = Bb // CH
    G = 8

    def body(rel_s, v1_ref, v2_ref, w_ref, b_ref, relt_ref, o_ref, r3, sems):
        g = pl.program_id(0)

        def cp(i, c):
            idx = rel_s[g * Bb + i]
            return pltpu.make_async_copy(
                relt_ref.at[pl.ds(idx * H, H)], r3.at[i], sems.at[c])

        for c in range(NCH):
            for j in range(CH):
                cp(c * CH + j, c).start()

        bf = jnp.bfloat16
        x1 = jnp.dot(v1_ref[...].astype(bf), w_ref[...],
                     preferred_element_type=jnp.float32) + b_ref[...]
        x2 = jnp.dot(v2_ref[...].astype(bf), w_ref[...],
                     preferred_element_type=jnp.float32) + b_ref[...]
        u1 = _logsig(x1).astype(bf)
        u2 = _logsig(x2).astype(bf).astype(jnp.float32)

        row = lax.broadcasted_iota(jnp.int32, (G, G * H), 0)
        col = lax.broadcasted_iota(jnp.int32, (G, G * H), 1)
        mask = (col // H) == row

        for c in range(NCH):
            for j in range(CH):
                cp(c * CH + j, c).wait()
            for q in range(CH // G):
                i0 = c * CH + q * G
                u1g = lax.slice_in_dim(u1, i0, i0 + G)
                u1t = jnp.concatenate([u1g] * G, axis=1)
                lhs = jnp.where(mask, u1t, jnp.zeros_like(u1t))
                rhs = r3[pl.ds(i0, G)].reshape(G * H, H)
                t = jnp.dot(lhs, rhs, preferred_element_type=jnp.float32)
                tb = t.astype(bf).astype(jnp.float32)
                u2g = lax.slice_in_dim(u2, i0, i0 + G)
                s = jnp.sum(tb * u2g, axis=1, keepdims=True)
                o_ref[pl.ds(i0, G), :] = 1.0 / (1.0 + jnp.exp(-s))

    grid_spec = pltpu.PrefetchScalarGridSpec(
        num_scalar_prefetch=1,
        grid=(NB,),
        in_specs=[
            pl.BlockSpec((Bb, E), lambda g, r: (g, 0)),
            pl.BlockSpec((Bb, E), lambda g, r: (g, 0)),
            pl.BlockSpec((E, H), lambda g, r: (0, 0)),
            pl.BlockSpec((1, H), lambda g, r: (0, 0)),
            pl.BlockSpec(memory_space=pl.ANY),
        ],
        out_specs=pl.BlockSpec((Bb, 1), lambda g, r: (g, 0)),
        scratch_shapes=[
            pltpu.VMEM((Bb, H, H), jnp.bfloat16),
            pltpu.SemaphoreType.DMA((NCH,)),
        ],
    )
    return pl.pallas_call(
        body, grid_spec=grid_spec,
        out_shape=jax.ShapeDtypeStruct((B, 1), jnp.float32),
    )(rel, v1, v2, wt, b2, relt)


def kernel(rel, sub, obj, embed_table, fc1_W, fc1_b, rel_table):
    B = rel.shape[0]
    hid = fc1_W.shape[0]
    idx_all = jnp.concatenate([sub, obj], axis=0).astype(jnp.int32)
    vsum = _sc_embed_pool(idx_all, embed_table)
    v1, v2 = vsum[:B], vsum[B:]
    wt = fc1_W.T.astype(jnp.bfloat16)
    b2 = fc1_b.reshape(1, hid)
    relt = rel_table.reshape(rel_table.shape[0] * hid, hid).astype(jnp.bfloat16)
    out = _tc_score(rel.astype(jnp.int32), v1, v2, wt, b2, relt, hid)
    return out.reshape(B, 1, 1)


# speedup vs baseline: 1.2735x; 1.2735x over previous
import functools

import jax
import jax.numpy as jnp
from jax import lax
from jax.experimental import pallas as pl
from jax.experimental.pallas import tpu as pltpu
from jax.experimental.pallas import tpu_sc as plsc


def _sc_embed_pool(idx_all, table):
    NI, L = idx_all.shape
    _, E = table.shape
    mesh = plsc.VectorSubcoreMesh(core_axis_name="c", subcore_axis_name="s")
    NW = mesh.num_cores * mesh.num_subcores
    IPW = NI // NW
    EC = E // 16
    offs = list(range(0, L - 15, 16))
    if L % 16:
        offs.append(L - 16)

    @functools.partial(
        pl.kernel,
        out_type=jax.ShapeDtypeStruct((NI, E), jnp.float32),
        mesh=mesh,
        scratch_types=[
            pltpu.VMEM((IPW + 2, L), jnp.int32),
            pltpu.VMEM((2, L, E), jnp.float32),
            pltpu.VMEM((IPW, E), jnp.float32),
            pltpu.SemaphoreType.DMA((2,)),
        ],
        compiler_params=pltpu.CompilerParams(use_tc_tiling_on_sc=False),
    )
    def k(idx_hbm, tab_hbm, out_hbm, idx_v, rows_v, out_v, sems):
        wid = lax.axis_index("s") * mesh.num_cores + lax.axis_index("c")
        base = wid * IPW
        zero16 = jnp.zeros((16,), jnp.int32)
        for r in (IPW, IPW + 1):
            for off in offs:
                idx_v[r, pl.ds(off, 16)] = zero16
        pltpu.sync_copy(idx_hbm.at[pl.ds(base, IPW)],
                        idx_v.at[pl.ds(0, IPW)])

        def start(i, buf):
            pltpu.async_copy(tab_hbm.at[idx_v.at[i]], rows_v.at[buf],
                             sems.at[buf])

        def wait(buf):
            pltpu.make_async_copy(tab_hbm.at[idx_v.at[0]], rows_v.at[buf],
                                  sems.at[buf]).wait()

        def accum(i, buf):
            acc = [rows_v[buf, 0, pl.ds(c * 16, 16)] for c in range(EC)]
            for l in range(1, L):
                for c in range(EC):
                    acc[c] = acc[c] + rows_v[buf, l, pl.ds(c * 16, 16)]
            for c in range(EC):
                out_v[i, pl.ds(c * 16, 16)] = acc[c]

        start(0, 0)

        @pl.loop(0, IPW, step=2)
        def _(i):
            start(i + 1, 1)
            wait(0)
            accum(i, 0)
            start(i + 2, 0)
            wait(1)
            accum(i + 1, 1)

        wait(0)
        pltpu.sync_copy(out_v, out_hbm.at[pl.ds(base, IPW)])

    return k(idx_all, table)


def _logsig(x):
    return jnp.minimum(x, 0.0) - jnp.log(1.0 + jnp.exp(-jnp.abs(x)))


def _tc_score(rel, v1, v2, wt, b2, relt, hid):
    B, E = v1.shape
    H = hid
    Bb = 256
    NB = B // Bb
    CH = 32
    NCH = Bb // CH
    G = 8

    def body(rel_s, v1_ref, v2_ref, w_ref, b_ref, relt_ref, o_ref, r3, sems):
        g = pl.program_id(0)

        def cp(i, c):
            idx = rel_s[g * Bb + i]
            return pltpu.make_async_copy(
                relt_ref.at[pl.ds(idx * H, H)], r3.at[i], sems.at[c])

        for j in range(CH):
            cp(j, 0).start()
        for j in range(CH):
            cp(CH + j, 1).start()

        bf = jnp.bfloat16
        x1 = jnp.dot(v1_ref[...].astype(bf), w_ref[...],
                     preferred_element_type=jnp.float32) + b_ref[...]
        x2 = jnp.dot(v2_ref[...].astype(bf), w_ref[...],
                     preferred_element_type=jnp.float32) + b_ref[...]
        u1 = _logsig(x1).astype(bf)
        u2 = _logsig(x2).astype(bf).astype(jnp.float32)

        row = lax.broadcasted_iota(jnp.int32, (G, G * H), 0)
        col = lax.broadcasted_iota(jnp.int32, (G, G * H), 1)
        mask = (col // H) == row

        for c in range(NCH):
            if c + 2 < NCH:
                for j in range(CH):
                    cp((c + 2) * CH + j, c + 2).start()
            for j in range(CH):
                cp(c * CH + j, c).wait()
            for q in range(CH // G):
                i0 = c * CH + q * G
                u1g = lax.slice_in_dim(u1, i0, i0 + G)
                u1t = jnp.concatenate([u1g] * G, axis=1)
                lhs = jnp.where(mask, u1t, jnp.zeros_like(u1t))
                rhs = r3[pl.ds(i0, G)].reshape(G * H, H)
                t = jnp.dot(lhs, rhs, preferred_element_type=jnp.float32)
                tb = t.astype(bf).astype(jnp.float32)
                u2g = lax.slice_in_dim(u2, i0, i0 + G)
                s = jnp.sum(tb * u2g, axis=1, keepdims=True)
                o_ref[pl.ds(i0, G), :] = 1.0 / (1.0 + jnp.exp(-s))

    grid_spec = pltpu.PrefetchScalarGridSpec(
        num_scalar_prefetch=1,
        grid=(NB,),
        in_specs=[
            pl.BlockSpec((Bb, E), lambda g, r: (g, 0)),
            pl.BlockSpec((Bb, E), lambda g, r: (g, 0)),
            pl.BlockSpec((E, H), lambda g, r: (0, 0)),
            pl.BlockSpec((1, H), lambda g, r: (0, 0)),
            pl.BlockSpec(memory_space=pl.ANY),
        ],
        out_specs=pl.BlockSpec((Bb, 1), lambda g, r: (g, 0)),
        scratch_shapes=[
            pltpu.VMEM((Bb, H, H), jnp.bfloat16),
            pltpu.SemaphoreType.DMA((NCH,)),
        ],
    )
    return pl.pallas_call(
        body, grid_spec=grid_spec,
        out_shape=jax.ShapeDtypeStruct((B, 1), jnp.float32),
    )(rel, v1, v2, wt, b2, relt)


def kernel(rel, sub, obj, embed_table, fc1_W, fc1_b, rel_table):
    B = rel.shape[0]
    hid = fc1_W.shape[0]
    idx_all = jnp.concatenate([sub, obj], axis=0).astype(jnp.int32)
    vsum = _sc_embed_pool(idx_all, embed_table)
    v1, v2 = vsum[:B], vsum[B:]
    wt = fc1_W.T.astype(jnp.bfloat16)
    b2 = fc1_b.reshape(1, hid)
    relt = rel_table.reshape(rel_table.shape[0] * hid, hid).astype(jnp.bfloat16)
    out = _tc_score(rel.astype(jnp.int32), v1, v2, wt, b2, relt, hid)
    return out.reshape(B, 1, 1)


# speedup vs baseline: 1.3175x; 1.0346x over previous
import functools

import jax
import jax.numpy as jnp
from jax import lax
from jax.experimental import pallas as pl
from jax.experimental.pallas import tpu as pltpu
from jax.experimental.pallas import tpu_sc as plsc


def _sc_embed_pool(idx_all, table):
    NI, L = idx_all.shape
    _, E = table.shape
    mesh = plsc.VectorSubcoreMesh(core_axis_name="c", subcore_axis_name="s")
    NW = mesh.num_cores * mesh.num_subcores
    IPW = NI // NW
    EC = E // 16
    offs = list(range(0, L - 15, 16))
    if L % 16:
        offs.append(L - 16)

    @functools.partial(
        pl.kernel,
        out_type=jax.ShapeDtypeStruct((NI, E), jnp.float32),
        mesh=mesh,
        scratch_types=[
            pltpu.VMEM((IPW + 2, L), jnp.int32),
            pltpu.VMEM((2, L, E), jnp.float32),
            pltpu.VMEM((IPW, E), jnp.float32),
            pltpu.SemaphoreType.DMA((2,)),
        ],
        compiler_params=pltpu.CompilerParams(use_tc_tiling_on_sc=False),
    )
    def k(idx_hbm, tab_hbm, out_hbm, idx_v, rows_v, out_v, sems):
        wid = lax.axis_index("s") * mesh.num_cores + lax.axis_index("c")
        base = wid * IPW
        zero16 = jnp.zeros((16,), jnp.int32)
        for r in (IPW, IPW + 1):
            for off in offs:
                idx_v[r, pl.ds(off, 16)] = zero16
        pltpu.sync_copy(idx_hbm.at[pl.ds(base, IPW)],
                        idx_v.at[pl.ds(0, IPW)])

        def start(i, buf):
            pltpu.async_copy(tab_hbm.at[idx_v.at[i]], rows_v.at[buf],
                             sems.at[buf])

        def wait(buf):
            pltpu.make_async_copy(tab_hbm.at[idx_v.at[0]], rows_v.at[buf],
                                  sems.at[buf]).wait()

        def accum(i, buf):
            acc = [rows_v[buf, 0, pl.ds(c * 16, 16)] for c in range(EC)]
            for l in range(1, L):
                for c in range(EC):
                    acc[c] = acc[c] + rows_v[buf, l, pl.ds(c * 16, 16)]
            for c in range(EC):
                out_v[i, pl.ds(c * 16, 16)] = acc[c]

        start(0, 0)

        @pl.loop(0, IPW, step=2)
        def _(i):
            start(i + 1, 1)
            wait(0)
            accum(i, 0)
            start(i + 2, 0)
            wait(1)
            accum(i + 1, 1)

        wait(0)
        pltpu.sync_copy(out_v, out_hbm.at[pl.ds(base, IPW)])

    return k(idx_all, table)


def _logsig(x):
    return jnp.minimum(x, 0.0) - jnp.log(1.0 + jnp.exp(-jnp.abs(x)))


def _tc_score(rel, v1, v2, wt, b2, relt, hid):
    B, E = v1.shape
    H = hid
    Bb = 256
    NB = B // Bb
    CH = 32
    NCH = Bb // CH
    G = 8

    def body(rel_s, v1_ref, v2_ref, w_ref, b_ref, relt_ref, o_ref, r3, sems):
        g = pl.program_id(0)

        def cp(i, c):
            idx = rel_s[g * Bb + i]
            return pltpu.make_async_copy(
                relt_ref.at[pl.ds(idx * H, H)],
                r3.at[pl.ds(i * H, H)], sems.at[c])

        def chunk_wait(c):
            pltpu.make_async_copy(
                relt_ref.at[pl.ds(0, CH * H)],
                r3.at[pl.ds(c * CH * H, CH * H)], sems.at[c]).wait()

        for c in range(NCH):
            for j in range(CH):
                cp(c * CH + j, c).start()

        bf = jnp.bfloat16
        x1 = jnp.dot(v1_ref[...].astype(bf), w_ref[...],
                     preferred_element_type=jnp.float32) + b_ref[...]
        x2 = jnp.dot(v2_ref[...].astype(bf), w_ref[...],
                     preferred_element_type=jnp.float32) + b_ref[...]
        u1 = _logsig(x1).astype(bf)
        u2 = _logsig(x2).astype(bf).astype(jnp.float32)

        row = lax.broadcasted_iota(jnp.int32, (G, G * H), 0)
        col = lax.broadcasted_iota(jnp.int32, (G, G * H), 1)
        mask = (col // H) == row

        for c in range(NCH):
            chunk_wait(c)
            for q in range(CH // G):
                i0 = c * CH + q * G
                u1g = lax.slice_in_dim(u1, i0, i0 + G)
                u1t = jnp.concatenate([u1g] * G, axis=1)
                lhs = jnp.where(mask, u1t, jnp.zeros_like(u1t))
                rhs = r3[pl.ds(i0 * H, G * H)]
                t = jnp.dot(lhs, rhs, preferred_element_type=jnp.float32)
                tb = t.astype(bf).astype(jnp.float32)
                u2g = lax.slice_in_dim(u2, i0, i0 + G)
                s = jnp.sum(tb * u2g, axis=1, keepdims=True)
                o_ref[pl.ds(i0, G), :] = 1.0 / (1.0 + jnp.exp(-s))

    grid_spec = pltpu.PrefetchScalarGridSpec(
        num_scalar_prefetch=1,
        grid=(NB,),
        in_specs=[
            pl.BlockSpec((Bb, E), lambda g, r: (g, 0)),
            pl.BlockSpec((Bb, E), lambda g, r: (g, 0)),
            pl.BlockSpec((E, H), lambda g, r: (0, 0)),
            pl.BlockSpec((1, H), lambda g, r: (0, 0)),
            pl.BlockSpec(memory_space=pl.ANY),
        ],
        out_specs=pl.BlockSpec((Bb, 1), lambda g, r: (g, 0)),
        scratch_shapes=[
            pltpu.VMEM((Bb * H, H), jnp.bfloat16),
            pltpu.SemaphoreType.DMA((NCH,)),
        ],
    )
    return pl.pallas_call(
        body, grid_spec=grid_spec,
        out_shape=jax.ShapeDtypeStruct((B, 1), jnp.float32),
    )(rel, v1, v2, wt, b2, relt)


def kernel(rel, sub, obj, embed_table, fc1_W, fc1_b, rel_table):
    B = rel.shape[0]
    hid = fc1_W.shape[0]
    idx_all = jnp.concatenate([sub, obj], axis=0).astype(jnp.int32)
    vsum = _sc_embed_pool(idx_all, embed_table)
    v1, v2 = vsum[:B], vsum[B:]
    wt = fc1_W.T.astype(jnp.bfloat16)
    b2 = fc1_b.reshape(1, hid)
    relt = rel_table.reshape(rel_table.shape[0] * hid, hid).astype(jnp.bfloat16)
    out = _tc_score(rel.astype(jnp.int32), v1, v2, wt, b2, relt, hid)
    return out.reshape(B, 1, 1)


# speedup vs baseline: 1.3827x; 1.0494x over previous
import functools

import jax
import jax.numpy as jnp
from jax import lax
from jax.experimental import pallas as pl
from jax.experimental.pallas import tpu as pltpu
from jax.experimental.pallas import tpu_sc as plsc


def _sc_embed_pool(idx_all, table):
    NI, L = idx_all.shape
    _, E = table.shape
    mesh = plsc.VectorSubcoreMesh(core_axis_name="c", subcore_axis_name="s")
    NW = mesh.num_cores * mesh.num_subcores
    IPW = NI // NW
    EC = E // 16
    offs = list(range(0, L - 15, 16))
    if L % 16:
        offs.append(L - 16)

    @functools.partial(
        pl.kernel,
        out_type=jax.ShapeDtypeStruct((NI, E), jnp.float32),
        mesh=mesh,
        scratch_types=[
            pltpu.VMEM((IPW + 2, L), jnp.int32),
            pltpu.VMEM((2, L, E), jnp.float32),
            pltpu.VMEM((IPW, E), jnp.float32),
            pltpu.SemaphoreType.DMA((2,)),
        ],
        compiler_params=pltpu.CompilerParams(use_tc_tiling_on_sc=False),
    )
    def k(idx_hbm, tab_hbm, out_hbm, idx_v, rows_v, out_v, sems):
        wid = lax.axis_index("s") * mesh.num_cores + lax.axis_index("c")
        base = wid * IPW
        zero16 = jnp.zeros((16,), jnp.int32)
        for r in (IPW, IPW + 1):
            for off in offs:
                idx_v[r, pl.ds(off, 16)] = zero16
        pltpu.sync_copy(idx_hbm.at[pl.ds(base, IPW)],
                        idx_v.at[pl.ds(0, IPW)])

        def start(i, buf):
            pltpu.async_copy(tab_hbm.at[idx_v.at[i]], rows_v.at[buf],
                             sems.at[buf])

        def wait(buf):
            pltpu.make_async_copy(tab_hbm.at[idx_v.at[0]], rows_v.at[buf],
                                  sems.at[buf]).wait()

        def accum(i, buf):
            acc = [rows_v[buf, 0, pl.ds(c * 16, 16)] for c in range(EC)]
            for l in range(1, L):
                for c in range(EC):
                    acc[c] = acc[c] + rows_v[buf, l, pl.ds(c * 16, 16)]
            for c in range(EC):
                out_v[i, pl.ds(c * 16, 16)] = acc[c]

        start(0, 0)

        @pl.loop(0, IPW, step=2)
        def _(i):
            start(i + 1, 1)
            wait(0)
            accum(i, 0)
            start(i + 2, 0)
            wait(1)
            accum(i + 1, 1)

        wait(0)
        pltpu.sync_copy(out_v, out_hbm.at[pl.ds(base, IPW)])

    return k(idx_all, table)


def _logsig(x):
    return jnp.minimum(x, 0.0) - jnp.log(1.0 + jnp.exp(-jnp.abs(x)))


def _tc_score(rel, v1, v2, wt, b2, relt, hid):
    B, E = v1.shape
    H = hid
    Bb = 512
    NB = B // Bb
    CH = 64
    NCH = Bb // CH
    G = 8

    def body(rel_s, v1_ref, v2_ref, w_ref, b_ref, relt_ref, o_ref, r3, sems):
        g = pl.program_id(0)

        def cp(i, c):
            idx = rel_s[g * Bb + i]
            return pltpu.make_async_copy(
                relt_ref.at[pl.ds(idx * H, H)],
                r3.at[pl.ds(i * H, H)], sems.at[c])

        def chunk_wait(c):
            pltpu.make_async_copy(
                relt_ref.at[pl.ds(0, CH * H)],
                r3.at[pl.ds(c * CH * H, CH * H)], sems.at[c]).wait()

        for c in range(NCH):
            for j in range(CH):
                cp(c * CH + j, c).start()

        bf = jnp.bfloat16
        x1 = jnp.dot(v1_ref[...].astype(bf), w_ref[...],
                     preferred_element_type=jnp.float32) + b_ref[...]
        x2 = jnp.dot(v2_ref[...].astype(bf), w_ref[...],
                     preferred_element_type=jnp.float32) + b_ref[...]
        u1 = _logsig(x1).astype(bf)
        u2 = _logsig(x2).astype(bf).astype(jnp.float32)

        row = lax.broadcasted_iota(jnp.int32, (G, G * H), 0)
        col = lax.broadcasted_iota(jnp.int32, (G, G * H), 1)
        mask = (col // H) == row

        for c in range(NCH):
            chunk_wait(c)
            for q in range(CH // G):
                i0 = c * CH + q * G
                u1g = lax.slice_in_dim(u1, i0, i0 + G)
                u1t = jnp.concatenate([u1g] * G, axis=1)
                lhs = jnp.where(mask, u1t, jnp.zeros_like(u1t))
                rhs = r3[pl.ds(i0 * H, G * H)]
                t = jnp.dot(lhs, rhs, preferred_element_type=jnp.float32)
                tb = t.astype(bf).astype(jnp.float32)
                u2g = lax.slice_in_dim(u2, i0, i0 + G)
                s = jnp.sum(tb * u2g, axis=1, keepdims=True)
                o_ref[pl.ds(i0, G), :] = 1.0 / (1.0 + jnp.exp(-s))

    grid_spec = pltpu.PrefetchScalarGridSpec(
        num_scalar_prefetch=1,
        grid=(NB,),
        in_specs=[
            pl.BlockSpec((Bb, E), lambda g, r: (g, 0)),
            pl.BlockSpec((Bb, E), lambda g, r: (g, 0)),
            pl.BlockSpec((E, H), lambda g, r: (0, 0)),
            pl.BlockSpec((1, H), lambda g, r: (0, 0)),
            pl.BlockSpec(memory_space=pl.ANY),
        ],
        out_specs=pl.BlockSpec((Bb, 1), lambda g, r: (g, 0)),
        scratch_shapes=[
            pltpu.VMEM((Bb * H, H), jnp.bfloat16),
            pltpu.SemaphoreType.DMA((NCH,)),
        ],
    )
    return pl.pallas_call(
        body, grid_spec=grid_spec,
        out_shape=jax.ShapeDtypeStruct((B, 1), jnp.float32),
    )(rel, v1, v2, wt, b2, relt)


def kernel(rel, sub, obj, embed_table, fc1_W, fc1_b, rel_table):
    B = rel.shape[0]
    hid = fc1_W.shape[0]
    idx_all = jnp.concatenate([sub, obj], axis=0).astype(jnp.int32)
    vsum = _sc_embed_pool(idx_all, embed_table)
    v1, v2 = vsum[:B], vsum[B:]
    wt = fc1_W.T.astype(jnp.bfloat16)
    b2 = fc1_b.reshape(1, hid)
    relt = rel_table.reshape(rel_table.shape[0] * hid, hid).astype(jnp.bfloat16)
    out = _tc_score(rel.astype(jnp.int32), v1, v2, wt, b2, relt, hid)
    return out.reshape(B, 1, 1)
